# baseline (speedup 1.0000x reference)
import dataclasses
import functools

import jax
import jax.numpy as jnp
import numpy as np
from jax import lax
from jax.experimental import pallas as pl
from jax.experimental.pallas import tpu as pltpu
from jax.experimental.pallas import tpu_sc as plsc

N_ATOMS = 10000
N_BONDS = 160000
MAX_NB = 16
D_ATOM = 133
D_BOND = 147
D_H = 600
DP = 768
PK = DP // 2

NC, NS = 2, 16
NW = NC * NS
NAP = 10240
BONDS_PER_W = N_BONDS // NW
GS_WIN = 8
GD_WIN = 40

GS_W0, GS_W1 = 640, 0
_SC0_ATOMS = NS * GS_W0
NAP_IDX = _SC0_ATOMS + (NS - 1) * GS_W1 + GS_W0

_PERM = np.concatenate([np.arange(0, DP, 2), np.arange(1, DP, 2)])


def _worker_id():
    return lax.axis_index("s") * NC + lax.axis_index("c")


CHUNKS = 5
CB = N_BONDS // CHUNKS
CB_W = CB // NW
_GD_NWIN = CB_W // GD_WIN
_GS_ROWS = GS_WIN * MAX_NB


@functools.cache
def _sc_kernels():
    mesh = plsc.VectorSubcoreMesh(core_axis_name="c", subcore_axis_name="s",
                                  num_cores=NC, num_subcores=NS)
    cp = pltpu.CompilerParams()
    if "needs_layout_passes" in pltpu.CompilerParams.__dataclass_fields__:
        cp = dataclasses.replace(cp, needs_layout_passes=False)

    @functools.partial(
        pl.kernel,
        out_type=jax.ShapeDtypeStruct((NAP, PK), jnp.float32),
        mesh=mesh,
        compiler_params=cp,
        scratch_types=[
            pltpu.VMEM((GS_W0 * MAX_NB,), jnp.int32),
            pltpu.VMEM((_GS_ROWS, PK), jnp.float32),
            pltpu.VMEM((_GS_ROWS, PK), jnp.float32),
            pltpu.VMEM((GS_WIN, PK), jnp.float32),
            pltpu.SemaphoreType.DMA,
            pltpu.SemaphoreType.DMA,
        ],
    )
    def sc_gather_sum(z_hbm, a2b_hbm, amsg_hbm, idxs, rows0, rows1, acc,
                      sem0, sem1):
        cidx = lax.axis_index("c")
        sidx = lax.axis_index("s")
        abase = jnp.where(cidx == 0, sidx * GS_W0,
                          _SC0_ATOMS + sidx * GS_W1)
        nwin2 = jnp.where(cidx == 0, GS_W0 // (2 * GS_WIN),
                          GS_W1 // (2 * GS_WIN))
        pltpu.sync_copy(a2b_hbm.at[pl.ds(abase * MAX_NB, GS_W0 * MAX_NB)],
                        idxs)
        zero = jnp.zeros((32,), jnp.bfloat16)

        def gather(w, rows, sem):
            return pltpu.make_async_copy(
                z_hbm.at[idxs.at[pl.ds(w * _GS_ROWS, _GS_ROWS)]], rows, sem)

        def compute_store(w, rows):
            for j in range(GS_WIN):
                r0 = j * MAX_NB

                @pl.loop(0, PK // 16)
                def _(ci, r0=r0, j=j):
                    c = pl.ds(ci * 16, 16)

                    def ld(k):
                        return plsc.bitcast(rows[r0 + k, c], jnp.bfloat16)

                    s0 = jnp.maximum(ld(0), zero)
                    s1 = jnp.maximum(ld(1), zero)
                    s2 = jnp.maximum(ld(2), zero)
                    s3 = jnp.maximum(ld(3), zero)
                    for k in range(4, MAX_NB, 4):
                        s0 = s0 + jnp.maximum(ld(k + 0), zero)
                        s1 = s1 + jnp.maximum(ld(k + 1), zero)
                        s2 = s2 + jnp.maximum(ld(k + 2), zero)
                        s3 = s3 + jnp.maximum(ld(k + 3), zero)
                    s = (s0 + s1) + (s2 + s3)
                    acc[j, c] = plsc.bitcast(s, jnp.float32)

            pltpu.sync_copy(acc, amsg_hbm.at[pl.ds(abase + w * GS_WIN,
                                                   GS_WIN)])

        @pl.when(nwin2 > 0)
        def _():
            gather(0, rows0, sem0).start()

        @pl.loop(0, nwin2)
        def _(w2):
            w = w2 * 2
            gather(w + 1, rows1, sem1).start()
            gather(w, rows0, sem0).wait()
            compute_store(w, rows0)

            @pl.when(w2 < nwin2 - 1)
            def _():
                gather(w + 2, rows0, sem0).start()

            gather(w + 1, rows1, sem1).wait()
            compute_store(w + 1, rows1)

    def make_gather2(c0):
        @functools.partial(
            pl.kernel,
            out_type=(jax.ShapeDtypeStruct((CB, PK), jnp.float32),
                      jax.ShapeDtypeStruct((CB, PK), jnp.float32)),
            mesh=mesh,
            compiler_params=cp,
            scratch_types=[
                pltpu.VMEM((CB_W,), jnp.int32),
                pltpu.VMEM((CB_W,), jnp.int32),
                pltpu.VMEM((GD_WIN, PK), jnp.float32),
                pltpu.VMEM((GD_WIN, PK), jnp.float32),
                pltpu.VMEM((GD_WIN, PK), jnp.float32),
                pltpu.VMEM((GD_WIN, PK), jnp.float32),
                pltpu.SemaphoreType.DMA,
                pltpu.SemaphoreType.DMA,
                pltpu.SemaphoreType.DMA,
                pltpu.SemaphoreType.DMA,
            ],
        )
        def sc_gather2(amsg_hbm, z_hbm, b2a_hbm, b2revb_hbm, g1_hbm, g2_hbm,
                       i1, i2, r1a, r2a, r1b, r2b, sga, sgb, swa, swb):
            wid = _worker_id()
            bin_ = c0 + wid * CB_W
            bout = wid * CB_W
            pltpu.sync_copy(b2a_hbm.at[pl.ds(bin_, CB_W)], i1)
            pltpu.sync_copy(b2revb_hbm.at[pl.ds(bin_, CB_W)], i2)

            def gathers(w, r1, r2, sem):
                s = pl.ds(w * GD_WIN, GD_WIN)
                c1 = pltpu.make_async_copy(amsg_hbm.at[i1.at[s]], r1, sem)
                c2 = pltpu.make_async_copy(z_hbm.at[i2.at[s]], r2, sem)
                return c1, c2

            def writebacks(w, r1, r2, sem):
                s = pl.ds(bout + w * GD_WIN, GD_WIN)
                c1 = pltpu.make_async_copy(r1, g1_hbm.at[s], sem)
                c2 = pltpu.make_async_copy(r2, g2_hbm.at[s], sem)
                return c1, c2

            def start2(pair):
                pair[0].start()
                pair[1].start()

            def wait2(pair):
                pair[0].wait()
                pair[1].wait()

            def step(w, r1, r2, sg, sw):
                wait2(gathers(w, r1, r2, sg))
                wb = writebacks(w, r1, r2, sw)
                start2(wb)
                wait2(wb)

            start2(gathers(0, r1a, r2a, sga))

            @pl.loop(0, _GD_NWIN // 2)
            def _(w2):
                w = w2 * 2
                start2(gathers(w + 1, r1b, r2b, sgb))
                step(w, r1a, r2a, sga, swa)

                @pl.when(w2 < (_GD_NWIN - 1) // 2)
                def _():
                    start2(gathers(w + 2, r1a, r2a, sga))

                step(w + 1, r1b, r2b, sgb, swb)

            step(_GD_NWIN - 1, r1a, r2a, sga, swa)

        return sc_gather2

    return sc_gather_sum, tuple(make_gather2(c * CB) for c in range(CHUNKS))


def _bf(x):
    return x.astype(jnp.bfloat16)


_HI = np.uint32(0xFFFF0000)
_RND = np.uint32(0x8000)


def _unpack(p):
    u = lax.bitcast_convert_type(p, jnp.uint32)
    fe = lax.bitcast_convert_type(u << 16, jnp.float32)
    fo = lax.bitcast_convert_type(u & _HI, jnp.float32)
    return fe, fo


def _pack(fe, fo):
    ue = lax.bitcast_convert_type(fe, jnp.uint32)
    uo = lax.bitcast_convert_type(fo, jnp.uint32)
    w = ((ue + _RND) >> 16) | ((uo + _RND) & _HI)
    return lax.bitcast_convert_type(w, jnp.float32)


def _mm_wi_body(fb_ref, wi_ref, out_ref):
    h = jnp.dot(_bf(fb_ref[...]), wi_ref[...],
                preferred_element_type=jnp.float32)
    out_ref[...] = _pack(h[:, :PK], h[:, PK:])


def _update_body(g1_ref, g2_ref, inp_ref, wht_ref, whb_ref, out_ref):
    g1e, g1o = _unpack(g1_ref[...])
    g2e, g2o = _unpack(g2_ref[...])
    de = g1e - jnp.maximum(g2e, 0.0)
    do = g1o - jnp.maximum(g2o, 0.0)
    h = jnp.dot(_bf(de), wht_ref[...], preferred_element_type=jnp.float32)
    h = h + jnp.dot(_bf(do), whb_ref[...], preferred_element_type=jnp.float32)
    ie, io = _unpack(inp_ref[...])
    out_ref[...] = _pack(ie + h[:, :PK], io + h[:, PK:])


def _update_body_car(g1_ref, g2_ref, inp_ref, wht_ref, whb_ref, car_ref,
                     out_ref):
    del car_ref
    _update_body(g1_ref, g2_ref, inp_ref, wht_ref, whb_ref, out_ref)


def _out_body(fa_ref, am_ref, wo1_ref, wo2t_ref, wo2b_ref, bo_ref, out_ref):
    ame, amo = _unpack(am_ref[...])
    h = jnp.dot(_bf(fa_ref[...]), wo1_ref[...],
                preferred_element_type=jnp.float32)
    h = h + jnp.dot(_bf(ame), wo2t_ref[...], preferred_element_type=jnp.float32)
    h = h + jnp.dot(_bf(amo), wo2b_ref[...], preferred_element_type=jnp.float32)
    out_ref[...] = jnp.maximum(h + bo_ref[...], 0.0)


_MB = 1280


def _mm_wi(f_bonds, wi_pp):
    return pl.pallas_call(
        _mm_wi_body,
        grid=(N_BONDS // _MB,),
        in_specs=[pl.BlockSpec((_MB, D_BOND), lambda i: (i, 0)),
                  pl.BlockSpec((D_BOND, DP), lambda i: (0, 0))],
        out_specs=pl.BlockSpec((_MB, PK), lambda i: (i, 0)),
        out_shape=jax.ShapeDtypeStruct((N_BONDS, PK), jnp.float32),
    )(f_bonds, wi_pp)


_CBLK = CB // _MB


def _update_chunk(c, g1c, g2c, inp, carrier, wht, whb):
    in_specs = [pl.BlockSpec((_MB, PK), lambda i: (i, 0)),
                pl.BlockSpec((_MB, PK), lambda i: (i, 0)),
                pl.BlockSpec((_MB, PK), lambda i, c=c: (c * _CBLK + i, 0)),
                pl.BlockSpec((PK, DP), lambda i: (0, 0)),
                pl.BlockSpec((PK, DP), lambda i: (0, 0))]
    args = [g1c, g2c, inp, wht, whb]
    aliases = {}
    body = _update_body
    if carrier is not None:
        in_specs.append(pl.BlockSpec((8, 128), lambda i: (0, 0)))
        args.append(carrier)
        aliases = {5: 0}
        body = _update_body_car
    return pl.pallas_call(
        body,
        grid=(_CBLK,),
        in_specs=in_specs,
        out_specs=pl.BlockSpec((_MB, PK), lambda i, c=c: (c * _CBLK + i, 0)),
        out_shape=jax.ShapeDtypeStruct((N_BONDS, PK), jnp.float32),
        input_output_aliases=aliases,
    )(*args)


_MA = 2000


def _mm_out(f_atoms, amsg, wo1, wo2t, wo2b, bo_row):
    return pl.pallas_call(
        _out_body,
        grid=(N_ATOMS // _MA,),
        in_specs=[pl.BlockSpec((_MA, D_ATOM), lambda i: (i, 0)),
                  pl.BlockSpec((_MA, PK), lambda i: (i, 0)),
                  pl.BlockSpec((D_ATOM, D_H), lambda i: (0, 0)),
                  pl.BlockSpec((PK, D_H), lambda i: (0, 0)),
                  pl.BlockSpec((PK, D_H), lambda i: (0, 0)),
                  pl.BlockSpec((1, D_H), lambda i: (0, 0))],
        out_specs=pl.BlockSpec((_MA, D_H), lambda i: (i, 0)),
        out_shape=jax.ShapeDtypeStruct((N_ATOMS, D_H), jnp.float32),
    )(f_atoms, amsg, wo1, wo2t, wo2b, bo_row)


def kernel(f_atoms, f_bonds, a2b, b2a, b2revb, W_i, W_h, W_o, b_o):
    pad_c = DP - D_H
    wi_pp = _bf(jnp.pad(W_i, ((0, 0), (0, pad_c)))[:, _PERM])
    wh_pp = jnp.pad(W_h, ((0, pad_c), (0, pad_c)))[_PERM][:, _PERM]
    wht = _bf(wh_pp[:PK])
    whb = _bf(wh_pp[PK:])
    wo2_pp = jnp.pad(W_o[D_ATOM:], ((0, pad_c), (0, 0)))[_PERM]
    wo2t = _bf(wo2_pp[:PK])
    wo2b = _bf(wo2_pp[PK:])
    wo1 = _bf(W_o[:D_ATOM])
    bo_row = b_o.reshape(1, D_H)
    a2b_flat = jnp.pad(a2b, ((0, NAP_IDX - N_ATOMS), (0, 0))).reshape(-1)

    sc_gather_sum, sc_gather2_chunks = _sc_kernels()
    inp = _mm_wi(f_bonds, wi_pp)

    z = inp
    for i in range(5):
        amsg = sc_gather_sum(z, a2b_flat)
        cur = None
        for c in range(CHUNKS):
            g1c, g2c = sc_gather2_chunks[c](amsg, z, b2a, b2revb)
            cur = _update_chunk(c, g1c, g2c, inp, cur, wht, whb)
        z = cur
    amsg = sc_gather_sum(z, a2b_flat)
    return _mm_out(f_atoms, amsg, wo1, wo2t, wo2b, bo_row)


# speedup vs baseline: 1.1012x; 1.1012x over previous
import dataclasses
import functools

import jax
import jax.numpy as jnp
import numpy as np
from jax import lax
from jax.experimental import pallas as pl
from jax.experimental.pallas import tpu as pltpu
from jax.experimental.pallas import tpu_sc as plsc

N_ATOMS = 10000
N_BONDS = 160000
MAX_NB = 16
D_ATOM = 133
D_BOND = 147
D_H = 600
DP = 768
PK = DP // 2

NC, NS = 2, 16
NW = NC * NS
NAP = 10240
BONDS_PER_W = N_BONDS // NW
GS_WIN = 8
GD_WIN = 40

GS_W0, GS_W1 = 512, 128
_SC0_ATOMS = NS * GS_W0
NAP_IDX = _SC0_ATOMS + (NS - 1) * GS_W1 + GS_W0

_PERM = np.concatenate([np.arange(0, DP, 2), np.arange(1, DP, 2)])


def _worker_id():
    return lax.axis_index("s") * NC + lax.axis_index("c")


CHUNKS = 5
CB = N_BONDS // CHUNKS
CB_W = CB // NW
_GD_NWIN = CB_W // GD_WIN
_GS_ROWS = GS_WIN * MAX_NB


@functools.cache
def _sc_kernels():
    mesh = plsc.VectorSubcoreMesh(core_axis_name="c", subcore_axis_name="s",
                                  num_cores=NC, num_subcores=NS)
    cp = pltpu.CompilerParams()
    if "needs_layout_passes" in pltpu.CompilerParams.__dataclass_fields__:
        cp = dataclasses.replace(cp, needs_layout_passes=False)

    @functools.partial(
        pl.kernel,
        out_type=jax.ShapeDtypeStruct((NAP, PK), jnp.float32),
        mesh=mesh,
        compiler_params=cp,
        scratch_types=[
            pltpu.VMEM((GS_W0 * MAX_NB,), jnp.int32),
            pltpu.VMEM((_GS_ROWS, PK), jnp.float32),
            pltpu.VMEM((_GS_ROWS, PK), jnp.float32),
            pltpu.VMEM((GS_WIN, PK), jnp.float32),
            pltpu.SemaphoreType.DMA,
            pltpu.SemaphoreType.DMA,
        ],
    )
    def sc_gather_sum(z_hbm, a2b_hbm, amsg_hbm, idxs, rows0, rows1, acc,
                      sem0, sem1):
        cidx = lax.axis_index("c")
        sidx = lax.axis_index("s")
        abase = jnp.where(cidx == 0, sidx * GS_W0,
                          _SC0_ATOMS + sidx * GS_W1)
        nwin2 = jnp.where(cidx == 0, GS_W0 // (2 * GS_WIN),
                          GS_W1 // (2 * GS_WIN))
        pltpu.sync_copy(a2b_hbm.at[pl.ds(abase * MAX_NB, GS_W0 * MAX_NB)],
                        idxs)
        zero = jnp.zeros((32,), jnp.bfloat16)

        def gather(w, rows, sem):
            return pltpu.make_async_copy(
                z_hbm.at[idxs.at[pl.ds(w * _GS_ROWS, _GS_ROWS)]], rows, sem)

        def compute_store(w, rows):
            for j in range(GS_WIN):
                r0 = j * MAX_NB

                @pl.loop(0, PK // 16)
                def _(ci, r0=r0, j=j):
                    c = pl.ds(ci * 16, 16)

                    def ld(k):
                        return plsc.bitcast(rows[r0 + k, c], jnp.bfloat16)

                    s0 = jnp.maximum(ld(0), zero)
                    s1 = jnp.maximum(ld(1), zero)
                    s2 = jnp.maximum(ld(2), zero)
                    s3 = jnp.maximum(ld(3), zero)
                    for k in range(4, MAX_NB, 4):
                        s0 = s0 + jnp.maximum(ld(k + 0), zero)
                        s1 = s1 + jnp.maximum(ld(k + 1), zero)
                        s2 = s2 + jnp.maximum(ld(k + 2), zero)
                        s3 = s3 + jnp.maximum(ld(k + 3), zero)
                    s = (s0 + s1) + (s2 + s3)
                    acc[j, c] = plsc.bitcast(s, jnp.float32)

            pltpu.sync_copy(acc, amsg_hbm.at[pl.ds(abase + w * GS_WIN,
                                                   GS_WIN)])

        @pl.when(nwin2 > 0)
        def _():
            gather(0, rows0, sem0).start()

        @pl.loop(0, nwin2)
        def _(w2):
            w = w2 * 2
            gather(w + 1, rows1, sem1).start()
            gather(w, rows0, sem0).wait()
            compute_store(w, rows0)

            @pl.when(w2 < nwin2 - 1)
            def _():
                gather(w + 2, rows0, sem0).start()

            gather(w + 1, rows1, sem1).wait()
            compute_store(w + 1, rows1)

    def make_gather2(c0):
        @functools.partial(
            pl.kernel,
            out_type=(jax.ShapeDtypeStruct((CB, PK), jnp.float32),
                      jax.ShapeDtypeStruct((CB, PK), jnp.float32)),
            mesh=mesh,
            compiler_params=cp,
            scratch_types=[
                pltpu.VMEM((CB_W,), jnp.int32),
                pltpu.VMEM((CB_W,), jnp.int32),
                pltpu.VMEM((GD_WIN, PK), jnp.float32),
                pltpu.VMEM((GD_WIN, PK), jnp.float32),
                pltpu.VMEM((GD_WIN, PK), jnp.float32),
                pltpu.VMEM((GD_WIN, PK), jnp.float32),
                pltpu.SemaphoreType.DMA,
                pltpu.SemaphoreType.DMA,
                pltpu.SemaphoreType.DMA,
                pltpu.SemaphoreType.DMA,
            ],
        )
        def sc_gather2(amsg_hbm, z_hbm, b2a_hbm, b2revb_hbm, g1_hbm, g2_hbm,
                       i1, i2, r1a, r2a, r1b, r2b, sga, sgb, swa, swb):
            wid = _worker_id()
            bin_ = c0 + wid * CB_W
            bout = wid * CB_W
            pltpu.sync_copy(b2a_hbm.at[pl.ds(bin_, CB_W)], i1)
            pltpu.sync_copy(b2revb_hbm.at[pl.ds(bin_, CB_W)], i2)

            def gathers(w, r1, r2, sem):
                s = pl.ds(w * GD_WIN, GD_WIN)
                c1 = pltpu.make_async_copy(amsg_hbm.at[i1.at[s]], r1, sem)
                c2 = pltpu.make_async_copy(z_hbm.at[i2.at[s]], r2, sem)
                return c1, c2

            def writebacks(w, r1, r2, sem):
                s = pl.ds(bout + w * GD_WIN, GD_WIN)
                c1 = pltpu.make_async_copy(r1, g1_hbm.at[s], sem)
                c2 = pltpu.make_async_copy(r2, g2_hbm.at[s], sem)
                return c1, c2

            def start2(pair):
                pair[0].start()
                pair[1].start()

            def wait2(pair):
                pair[0].wait()
                pair[1].wait()

            def step(w, r1, r2, sg, sw):
                wait2(gathers(w, r1, r2, sg))
                wb = writebacks(w, r1, r2, sw)
                start2(wb)
                wait2(wb)

            start2(gathers(0, r1a, r2a, sga))

            @pl.loop(0, _GD_NWIN // 2)
            def _(w2):
                w = w2 * 2
                start2(gathers(w + 1, r1b, r2b, sgb))
                step(w, r1a, r2a, sga, swa)

                @pl.when(w2 < (_GD_NWIN - 1) // 2)
                def _():
                    start2(gathers(w + 2, r1a, r2a, sga))

                step(w + 1, r1b, r2b, sgb, swb)

            step(_GD_NWIN - 1, r1a, r2a, sga, swa)

        return sc_gather2

    return sc_gather_sum, tuple(make_gather2(c * CB) for c in range(CHUNKS))


def _bf(x):
    return x.astype(jnp.bfloat16)


_HI = np.uint32(0xFFFF0000)
_RND = np.uint32(0x8000)


def _unpack(p):
    u = lax.bitcast_convert_type(p, jnp.uint32)
    fe = lax.bitcast_convert_type(u << 16, jnp.float32)
    fo = lax.bitcast_convert_type(u & _HI, jnp.float32)
    return fe, fo


def _pack(fe, fo):
    ue = lax.bitcast_convert_type(fe, jnp.uint32)
    uo = lax.bitcast_convert_type(fo, jnp.uint32)
    w = ((ue + _RND) >> 16) | ((uo + _RND) & _HI)
    return lax.bitcast_convert_type(w, jnp.float32)


def _mm_wi_body(fb_ref, wi_ref, out_ref):
    h = jnp.dot(_bf(fb_ref[...]), wi_ref[...],
                preferred_element_type=jnp.float32)
    out_ref[...] = _pack(h[:, :PK], h[:, PK:])


def _update_body(g1_ref, g2_ref, inp_ref, wht_ref, whb_ref, out_ref):
    g1e, g1o = _unpack(g1_ref[...])
    g2e, g2o = _unpack(g2_ref[...])
    de = g1e - jnp.maximum(g2e, 0.0)
    do = g1o - jnp.maximum(g2o, 0.0)
    h = jnp.dot(_bf(de), wht_ref[...], preferred_element_type=jnp.float32)
    h = h + jnp.dot(_bf(do), whb_ref[...], preferred_element_type=jnp.float32)
    ie, io = _unpack(inp_ref[...])
    out_ref[...] = _pack(ie + h[:, :PK], io + h[:, PK:])


def _update_body_car(g1_ref, g2_ref, inp_ref, wht_ref, whb_ref, car_ref,
                     out_ref):
    del car_ref
    _update_body(g1_ref, g2_ref, inp_ref, wht_ref, whb_ref, out_ref)


def _out_body(fa_ref, am_ref, wo1_ref, wo2t_ref, wo2b_ref, bo_ref, out_ref):
    ame, amo = _unpack(am_ref[...])
    h = jnp.dot(_bf(fa_ref[...]), wo1_ref[...],
                preferred_element_type=jnp.float32)
    h = h + jnp.dot(_bf(ame), wo2t_ref[...], preferred_element_type=jnp.float32)
    h = h + jnp.dot(_bf(amo), wo2b_ref[...], preferred_element_type=jnp.float32)
    out_ref[...] = jnp.maximum(h + bo_ref[...], 0.0)


_MB = 1280


def _mm_wi(f_bonds, wi_pp):
    return pl.pallas_call(
        _mm_wi_body,
        grid=(N_BONDS // _MB,),
        in_specs=[pl.BlockSpec((_MB, D_BOND), lambda i: (i, 0)),
                  pl.BlockSpec((D_BOND, DP), lambda i: (0, 0))],
        out_specs=pl.BlockSpec((_MB, PK), lambda i: (i, 0)),
        out_shape=jax.ShapeDtypeStruct((N_BONDS, PK), jnp.float32),
    )(f_bonds, wi_pp)


_CBLK = CB // _MB


def _update_chunk(c, g1c, g2c, inp, carrier, wht, whb):
    in_specs = [pl.BlockSpec((_MB, PK), lambda i: (i, 0)),
                pl.BlockSpec((_MB, PK), lambda i: (i, 0)),
                pl.BlockSpec((_MB, PK), lambda i, c=c: (c * _CBLK + i, 0)),
                pl.BlockSpec((PK, DP), lambda i: (0, 0)),
                pl.BlockSpec((PK, DP), lambda i: (0, 0))]
    args = [g1c, g2c, inp, wht, whb]
    aliases = {}
    body = _update_body
    if carrier is not None:
        in_specs.append(pl.BlockSpec((8, 128), lambda i: (0, 0)))
        args.append(carrier)
        aliases = {5: 0}
        body = _update_body_car
    return pl.pallas_call(
        body,
        grid=(_CBLK,),
        in_specs=in_specs,
        out_specs=pl.BlockSpec((_MB, PK), lambda i, c=c: (c * _CBLK + i, 0)),
        out_shape=jax.ShapeDtypeStruct((N_BONDS, PK), jnp.float32),
        input_output_aliases=aliases,
    )(*args)


_MA = 2000


def _mm_out(f_atoms, amsg, wo1, wo2t, wo2b, bo_row):
    return pl.pallas_call(
        _out_body,
        grid=(N_ATOMS // _MA,),
        in_specs=[pl.BlockSpec((_MA, D_ATOM), lambda i: (i, 0)),
                  pl.BlockSpec((_MA, PK), lambda i: (i, 0)),
                  pl.BlockSpec((D_ATOM, D_H), lambda i: (0, 0)),
                  pl.BlockSpec((PK, D_H), lambda i: (0, 0)),
                  pl.BlockSpec((PK, D_H), lambda i: (0, 0)),
                  pl.BlockSpec((1, D_H), lambda i: (0, 0))],
        out_specs=pl.BlockSpec((_MA, D_H), lambda i: (i, 0)),
        out_shape=jax.ShapeDtypeStruct((N_ATOMS, D_H), jnp.float32),
    )(f_atoms, amsg, wo1, wo2t, wo2b, bo_row)


def kernel(f_atoms, f_bonds, a2b, b2a, b2revb, W_i, W_h, W_o, b_o):
    pad_c = DP - D_H
    wi_pp = _bf(jnp.pad(W_i, ((0, 0), (0, pad_c)))[:, _PERM])
    wh_pp = jnp.pad(W_h, ((0, pad_c), (0, pad_c)))[_PERM][:, _PERM]
    wht = _bf(wh_pp[:PK])
    whb = _bf(wh_pp[PK:])
    wo2_pp = jnp.pad(W_o[D_ATOM:], ((0, pad_c), (0, 0)))[_PERM]
    wo2t = _bf(wo2_pp[:PK])
    wo2b = _bf(wo2_pp[PK:])
    wo1 = _bf(W_o[:D_ATOM])
    bo_row = b_o.reshape(1, D_H)
    a2b_flat = jnp.pad(a2b, ((0, NAP_IDX - N_ATOMS), (0, 0))).reshape(-1)

    sc_gather_sum, sc_gather2_chunks = _sc_kernels()
    inp = _mm_wi(f_bonds, wi_pp)

    z = inp
    for i in range(5):
        amsg = sc_gather_sum(z, a2b_flat)
        cur = None
        for c in range(CHUNKS):
            g1c, g2c = sc_gather2_chunks[c](amsg, z, b2a, b2revb)
            cur = _update_chunk(c, g1c, g2c, inp, cur, wht, whb)
        z = cur
    amsg = sc_gather_sum(z, a2b_flat)
    return _mm_out(f_atoms, amsg, wo1, wo2t, wo2b, bo_row)
